# Initial kernel scaffold; baseline (speedup 1.0000x reference)
import jax
import jax.numpy as jnp
from jax.experimental import pallas as pl


def kernel(x, seq, pause, edge_index, W_x, b_x, g_x, be_x, W_p, b_p, g_p, be_p, W_e, b_e, g_e, be_e, W_f, b_f, g_f, be_f, W_l, b_l, W_r, g_c, be_c, W_o, b_o):
    raise NotImplementedError("write your pallas kernel here")


# speedup vs baseline: 3.3577x; 3.3577x over previous
import functools

import jax
import jax.numpy as jnp
from jax import lax
from jax.experimental import pallas as pl
from jax.experimental.pallas import tpu as pltpu
from jax.experimental.pallas import tpu_sc as plsc

N = 10000
E = 160000
SEQ_D = 9216
H = 32
EPS = 1e-5

BLK = 400
CHUNK = 128
NW = 32
NCHUNK = E // CHUNK
ROWS_PER_S = 624
ROWS_LAST = N - 15 * ROWS_PER_S


def _gelu(x):
    return 0.5 * x * (1.0 + lax.erf(x * (2.0 ** -0.5)))


def _bn_cols(a, g, b):
    m = jnp.mean(a, axis=0, keepdims=True)
    v = jnp.mean((a - m) ** 2, axis=0, keepdims=True)
    return (a - m) / jnp.sqrt(v + EPS) * g + b


def _gemm_body(seq_ref, we_ref, a_ref):
    a_ref[...] = jnp.dot(seq_ref[...], we_ref[...],
                         preferred_element_type=jnp.float32)


def _encoder_gemm(seq, W_e):
    return pl.pallas_call(
        _gemm_body,
        grid=(N // BLK,),
        in_specs=[
            pl.BlockSpec((BLK, SEQ_D), lambda i: (i, 0)),
            pl.BlockSpec((SEQ_D, H), lambda i: (0, 0)),
        ],
        out_specs=pl.BlockSpec((BLK, H), lambda i: (i, 0)),
        out_shape=jax.ShapeDtypeStruct((N, H), jnp.float32),
    )(seq, W_e)


def _mid_body(a_ref, x_ref, p_ref, wx_ref, gx_ref, bex_ref, wp_ref, gp_ref,
              bep_ref, ge_ref, bee_ref, wf_ref, gf_ref, bef_ref, h2_ref):
    A = a_ref[...]
    h_e = _gelu(_bn_cols(A, ge_ref[...], bee_ref[...]))

    xv = x_ref[...]
    mx = jnp.mean(xv, axis=0, keepdims=True)
    vx = jnp.mean((xv - mx) ** 2, axis=0, keepdims=True)
    wx = wx_ref[...]
    h_x = _gelu((xv - mx) * wx / jnp.sqrt(vx * wx * wx + EPS)
                * gx_ref[...] + bex_ref[...])

    pv = p_ref[...]
    mp = jnp.mean(pv, axis=0, keepdims=True)
    vp = jnp.mean((pv - mp) ** 2, axis=0, keepdims=True)
    wp = wp_ref[...]
    h_p = _gelu((pv - mp) * wp / jnp.sqrt(vp * wp * wp + EPS)
                * gp_ref[...] + bep_ref[...])

    cat = jnp.concatenate([h_x + h_e, h_p], axis=1)
    B = jnp.dot(cat, wf_ref[...], preferred_element_type=jnp.float32)
    h2_ref[...] = _gelu(_bn_cols(B, gf_ref[...], bef_ref[...]))


def _mid_stage(A, x, pause1, W_x, g_x, be_x, W_p, g_p, be_p, g_e, be_e,
               W_f, g_f, be_f):
    return pl.pallas_call(
        _mid_body,
        out_shape=jax.ShapeDtypeStruct((N, H), jnp.float32),
    )(A, x, pause1, W_x, g_x, be_x, W_p, g_p, be_p, g_e, be_e, W_f, g_f, be_f)


def _sc_scatter(h2, src, dst, zeros):
    mesh = plsc.VectorSubcoreMesh(core_axis_name="c", subcore_axis_name="s")

    @functools.partial(
        pl.kernel,
        out_type=(jax.ShapeDtypeStruct((N, H), jnp.float32),
                  jax.ShapeDtypeStruct((N, H), jnp.float32)),
        mesh=mesh,
        scratch_types=[
            pltpu.VMEM((CHUNK,), jnp.int32),
            pltpu.VMEM((CHUNK,), jnp.int32),
            pltpu.VMEM((CHUNK, H), jnp.float32),
            pltpu.VMEM_SHARED((N, H), jnp.float32),
            pltpu.SemaphoreType.DMA,
        ],
        compiler_params=pltpu.CompilerParams(use_tc_tiling_on_sc=False),
    )
    def k(h2_hbm, src_hbm, dst_hbm, zeros_hbm, out0_hbm, out1_hbm,
          idx_v, dst_v, rows_v, agg_sh, sem):
        c = lax.axis_index("c")
        s = lax.axis_index("s")
        w = c * 16 + s

        def _row_copy(src_ref, dst_ref):
            off = pl.multiple_of(s * ROWS_PER_S, 8)

            @pl.when(s < 15)
            def _():
                pltpu.sync_copy(src_ref.at[pl.ds(off, ROWS_PER_S)],
                                dst_ref.at[pl.ds(off, ROWS_PER_S)])

            @pl.when(s == 15)
            def _():
                pltpu.sync_copy(src_ref.at[pl.ds(15 * ROWS_PER_S, ROWS_LAST)],
                                dst_ref.at[pl.ds(15 * ROWS_PER_S, ROWS_LAST)])

        _row_copy(zeros_hbm, agg_sh)
        plsc.subcore_barrier()

        def step(t):
            base = pl.multiple_of(t * CHUNK, CHUNK)
            pltpu.sync_copy(src_hbm.at[pl.ds(base, CHUNK)], idx_v)
            pltpu.async_copy(h2_hbm.at[idx_v], rows_v, sem).wait()
            pltpu.sync_copy(dst_hbm.at[pl.ds(base, CHUNK)], dst_v)
            pltpu.sync_copy(rows_v, agg_sh.at[dst_v], add=True)

        nfull = NCHUNK // NW
        rem = NCHUNK - nfull * NW

        def body(j, carry):
            step(j * NW + w)
            return carry
        lax.fori_loop(0, nfull, body, 0)

        @pl.when(w < rem)
        def _():
            step(nfull * NW + w)

        plsc.subcore_barrier()

        @pl.when(c == 0)
        def _():
            _row_copy(agg_sh, out0_hbm)

        @pl.when(c == 1)
        def _():
            _row_copy(agg_sh, out1_hbm)

    return k(h2, src, dst, zeros)


def _out_body(agg0_ref, agg1_ref, h2_ref, wl_ref, wr_ref, gc_ref, bec_ref,
              wo_ref, bo_ref, z_ref, out_ref):
    agg = agg0_ref[...] + agg1_ref[...]
    h2 = h2_ref[...]
    z0 = (jnp.dot(agg, wl_ref[...], preferred_element_type=jnp.float32)
          + jnp.dot(h2, wr_ref[...], preferred_element_type=jnp.float32))
    z = _gelu(_bn_cols(z0, gc_ref[...], bec_ref[...]))
    z_ref[...] = z
    out_ref[...] = jnp.maximum(
        jnp.dot(z, wo_ref[...], preferred_element_type=jnp.float32)
        + bo_ref[...], 0.0)


def _out_stage(agg0, agg1, h2, W_l, W_r, g_c, be_c, W_o, b_o):
    return pl.pallas_call(
        _out_body,
        out_shape=(jax.ShapeDtypeStruct((N, H), jnp.float32),
                   jax.ShapeDtypeStruct((N, 1), jnp.float32)),
    )(agg0, agg1, h2, W_l, W_r, g_c, be_c, W_o, b_o)


def kernel(x, seq, pause, edge_index, W_x, b_x, g_x, be_x, W_p, b_p, g_p, be_p,
           W_e, b_e, g_e, be_e, W_f, b_f, g_f, be_f, W_l, b_l, W_r, g_c, be_c,
           W_o, b_o):
    A = _encoder_gemm(seq, W_e)

    h2 = _mid_stage(
        A, x, pause.reshape(N, 1),
        W_x, g_x.reshape(1, H), be_x.reshape(1, H),
        W_p, g_p.reshape(1, H), be_p.reshape(1, H),
        g_e.reshape(1, H), be_e.reshape(1, H),
        W_f, g_f.reshape(1, H), be_f.reshape(1, H))

    src = edge_index[0]
    dst = edge_index[1]
    zeros = jnp.zeros((N, H), jnp.float32)
    agg0, agg1 = _sc_scatter(h2, src, dst, zeros)

    z, out = _out_stage(agg0, agg1, h2, W_l, W_r,
                        g_c.reshape(1, H), be_c.reshape(1, H),
                        W_o, b_o.reshape(1, 1))
    return (out, z)


# speedup vs baseline: 3.3612x; 1.0010x over previous
import functools

import jax
import jax.numpy as jnp
from jax import lax
from jax.experimental import pallas as pl
from jax.experimental.pallas import tpu as pltpu
from jax.experimental.pallas import tpu_sc as plsc

N = 10000
E = 160000
SEQ_D = 9216
H = 32
EPS = 1e-5

BLK = 400
CHUNK = 128
NW = 32
NCHUNK = E // CHUNK
ROWS_PER_S = 624
ROWS_LAST = N - 15 * ROWS_PER_S


def _gelu(x):
    return 0.5 * x * (1.0 + lax.erf(x * (2.0 ** -0.5)))


def _bn_cols(a, g, b):
    m = jnp.mean(a, axis=0, keepdims=True)
    v = jnp.mean((a - m) ** 2, axis=0, keepdims=True)
    return (a - m) / jnp.sqrt(v + EPS) * g + b


def _gemm_body(seq_ref, we_ref, a_ref):
    a_ref[...] = jnp.dot(seq_ref[...].astype(jnp.bfloat16),
                         we_ref[...].astype(jnp.bfloat16),
                         preferred_element_type=jnp.float32)


def _encoder_gemm(seq, W_e):
    return pl.pallas_call(
        _gemm_body,
        grid=(N // BLK,),
        in_specs=[
            pl.BlockSpec((BLK, SEQ_D), lambda i: (i, 0)),
            pl.BlockSpec((SEQ_D, H), lambda i: (0, 0)),
        ],
        out_specs=pl.BlockSpec((BLK, H), lambda i: (i, 0)),
        out_shape=jax.ShapeDtypeStruct((N, H), jnp.float32),
    )(seq, W_e)


def _mid_body(a_ref, x_ref, p_ref, wx_ref, gx_ref, bex_ref, wp_ref, gp_ref,
              bep_ref, ge_ref, bee_ref, wf_ref, gf_ref, bef_ref, h2_ref):
    A = a_ref[...]
    h_e = _gelu(_bn_cols(A, ge_ref[...], bee_ref[...]))

    xv = x_ref[...]
    mx = jnp.mean(xv, axis=0, keepdims=True)
    vx = jnp.mean((xv - mx) ** 2, axis=0, keepdims=True)
    wx = wx_ref[...]
    h_x = _gelu((xv - mx) * wx / jnp.sqrt(vx * wx * wx + EPS)
                * gx_ref[...] + bex_ref[...])

    pv = p_ref[...]
    mp = jnp.mean(pv, axis=0, keepdims=True)
    vp = jnp.mean((pv - mp) ** 2, axis=0, keepdims=True)
    wp = wp_ref[...]
    h_p = _gelu((pv - mp) * wp / jnp.sqrt(vp * wp * wp + EPS)
                * gp_ref[...] + bep_ref[...])

    cat = jnp.concatenate([h_x + h_e, h_p], axis=1)
    B = jnp.dot(cat, wf_ref[...], preferred_element_type=jnp.float32)
    h2_ref[...] = _gelu(_bn_cols(B, gf_ref[...], bef_ref[...]))


def _mid_stage(A, x, pause1, W_x, g_x, be_x, W_p, g_p, be_p, g_e, be_e,
               W_f, g_f, be_f):
    return pl.pallas_call(
        _mid_body,
        out_shape=jax.ShapeDtypeStruct((N, H), jnp.float32),
    )(A, x, pause1, W_x, g_x, be_x, W_p, g_p, be_p, g_e, be_e, W_f, g_f, be_f)


def _sc_scatter(h2, src, dst, zeros):
    mesh = plsc.VectorSubcoreMesh(core_axis_name="c", subcore_axis_name="s")

    @functools.partial(
        pl.kernel,
        out_type=(jax.ShapeDtypeStruct((N, H), jnp.float32),
                  jax.ShapeDtypeStruct((N, H), jnp.float32)),
        mesh=mesh,
        scratch_types=[
            pltpu.VMEM((CHUNK,), jnp.int32),
            pltpu.VMEM((CHUNK,), jnp.int32),
            pltpu.VMEM((CHUNK, H), jnp.float32),
            pltpu.VMEM_SHARED((N, H), jnp.float32),
            pltpu.SemaphoreType.DMA,
        ],
        compiler_params=pltpu.CompilerParams(use_tc_tiling_on_sc=False),
    )
    def k(h2_hbm, src_hbm, dst_hbm, zeros_hbm, out0_hbm, out1_hbm,
          idx_v, dst_v, rows_v, agg_sh, sem):
        c = lax.axis_index("c")
        s = lax.axis_index("s")
        w = c * 16 + s

        def _row_copy(src_ref, dst_ref):
            off = pl.multiple_of(s * ROWS_PER_S, 8)

            @pl.when(s < 15)
            def _():
                pltpu.sync_copy(src_ref.at[pl.ds(off, ROWS_PER_S)],
                                dst_ref.at[pl.ds(off, ROWS_PER_S)])

            @pl.when(s == 15)
            def _():
                pltpu.sync_copy(src_ref.at[pl.ds(15 * ROWS_PER_S, ROWS_LAST)],
                                dst_ref.at[pl.ds(15 * ROWS_PER_S, ROWS_LAST)])

        _row_copy(zeros_hbm, agg_sh)
        plsc.subcore_barrier()

        def step(t):
            base = pl.multiple_of(t * CHUNK, CHUNK)
            pltpu.sync_copy(src_hbm.at[pl.ds(base, CHUNK)], idx_v)
            pltpu.async_copy(h2_hbm.at[idx_v], rows_v, sem).wait()
            pltpu.sync_copy(dst_hbm.at[pl.ds(base, CHUNK)], dst_v)
            pltpu.sync_copy(rows_v, agg_sh.at[dst_v], add=True)

        nfull = NCHUNK // NW
        rem = NCHUNK - nfull * NW

        def body(j, carry):
            step(j * NW + w)
            return carry
        lax.fori_loop(0, nfull, body, 0)

        @pl.when(w < rem)
        def _():
            step(nfull * NW + w)

        plsc.subcore_barrier()

        @pl.when(c == 0)
        def _():
            _row_copy(agg_sh, out0_hbm)

        @pl.when(c == 1)
        def _():
            _row_copy(agg_sh, out1_hbm)

    return k(h2, src, dst, zeros)


def _out_body(agg0_ref, agg1_ref, h2_ref, wl_ref, wr_ref, gc_ref, bec_ref,
              wo_ref, bo_ref, z_ref, out_ref):
    agg = agg0_ref[...] + agg1_ref[...]
    h2 = h2_ref[...]
    z0 = (jnp.dot(agg, wl_ref[...], preferred_element_type=jnp.float32)
          + jnp.dot(h2, wr_ref[...], preferred_element_type=jnp.float32))
    z = _gelu(_bn_cols(z0, gc_ref[...], bec_ref[...]))
    z_ref[...] = z
    out_ref[...] = jnp.maximum(
        jnp.dot(z, wo_ref[...], preferred_element_type=jnp.float32)
        + bo_ref[...], 0.0)


def _out_stage(agg0, agg1, h2, W_l, W_r, g_c, be_c, W_o, b_o):
    return pl.pallas_call(
        _out_body,
        out_shape=(jax.ShapeDtypeStruct((N, H), jnp.float32),
                   jax.ShapeDtypeStruct((N, 1), jnp.float32)),
    )(agg0, agg1, h2, W_l, W_r, g_c, be_c, W_o, b_o)


def kernel(x, seq, pause, edge_index, W_x, b_x, g_x, be_x, W_p, b_p, g_p, be_p,
           W_e, b_e, g_e, be_e, W_f, b_f, g_f, be_f, W_l, b_l, W_r, g_c, be_c,
           W_o, b_o):
    A = _encoder_gemm(seq, W_e)

    h2 = _mid_stage(
        A, x, pause.reshape(N, 1),
        W_x, g_x.reshape(1, H), be_x.reshape(1, H),
        W_p, g_p.reshape(1, H), be_p.reshape(1, H),
        g_e.reshape(1, H), be_e.reshape(1, H),
        W_f, g_f.reshape(1, H), be_f.reshape(1, H))

    src = edge_index[0]
    dst = edge_index[1]
    zeros = jnp.zeros((N, H), jnp.float32)
    agg0, agg1 = _sc_scatter(h2, src, dst, zeros)

    z, out = _out_stage(agg0, agg1, h2, W_l, W_r,
                        g_c.reshape(1, H), be_c.reshape(1, H),
                        W_o, b_o.reshape(1, 1))
    return (out, z)


# speedup vs baseline: 4.1786x; 1.2432x over previous
import functools

import jax
import jax.numpy as jnp
from jax import lax
from jax.experimental import pallas as pl
from jax.experimental.pallas import tpu as pltpu
from jax.experimental.pallas import tpu_sc as plsc

N = 10000
E = 160000
SEQ_D = 9216
H = 32
EPS = 1e-5

BLK = 400
CHUNK = 1000
NW = 32
NCHUNK = E // CHUNK
ROWS_PER_S = 624
ROWS_LAST = N - 15 * ROWS_PER_S


def _gelu(x):
    return 0.5 * x * (1.0 + lax.erf(x * (2.0 ** -0.5)))


def _bn_cols(a, g, b):
    m = jnp.mean(a, axis=0, keepdims=True)
    v = jnp.mean((a - m) ** 2, axis=0, keepdims=True)
    return (a - m) / jnp.sqrt(v + EPS) * g + b


def _gemm_body(seq_ref, we_ref, a_ref):
    a_ref[...] = jnp.dot(seq_ref[...].astype(jnp.bfloat16),
                         we_ref[...].astype(jnp.bfloat16),
                         preferred_element_type=jnp.float32)


def _encoder_gemm(seq, W_e):
    return pl.pallas_call(
        _gemm_body,
        grid=(N // BLK,),
        in_specs=[
            pl.BlockSpec((BLK, SEQ_D), lambda i: (i, 0)),
            pl.BlockSpec((SEQ_D, H), lambda i: (0, 0)),
        ],
        out_specs=pl.BlockSpec((BLK, H), lambda i: (i, 0)),
        out_shape=jax.ShapeDtypeStruct((N, H), jnp.float32),
    )(seq, W_e)


def _mid_body(a_ref, x_ref, p_ref, wx_ref, gx_ref, bex_ref, wp_ref, gp_ref,
              bep_ref, ge_ref, bee_ref, wf_ref, gf_ref, bef_ref, h2_ref):
    def v(ref):
        return ref[...].reshape(1, H)
    A = a_ref[...]
    h_e = _gelu(_bn_cols(A, v(ge_ref), v(bee_ref)))

    xv = x_ref[...]
    mx = jnp.mean(xv, axis=0, keepdims=True)
    vx = jnp.mean((xv - mx) ** 2, axis=0, keepdims=True)
    wx = wx_ref[...]
    h_x = _gelu((xv - mx) * wx / jnp.sqrt(vx * wx * wx + EPS)
                * v(gx_ref) + v(bex_ref))

    pv = p_ref[...]
    mp = jnp.mean(pv, axis=0, keepdims=True)
    vp = jnp.mean((pv - mp) ** 2, axis=0, keepdims=True)
    wp = wp_ref[...]
    h_p = _gelu((pv - mp) * wp / jnp.sqrt(vp * wp * wp + EPS)
                * v(gp_ref) + v(bep_ref))

    cat = jnp.concatenate([h_x + h_e, h_p], axis=1)
    B = jnp.dot(cat, wf_ref[...], preferred_element_type=jnp.float32)
    h2_ref[...] = _gelu(_bn_cols(B, v(gf_ref), v(bef_ref)))


def _mid_stage(A, x, pause1, W_x, g_x, be_x, W_p, g_p, be_p, g_e, be_e,
               W_f, g_f, be_f):
    return pl.pallas_call(
        _mid_body,
        out_shape=jax.ShapeDtypeStruct((N, H), jnp.float32),
    )(A, x, pause1, W_x, g_x, be_x, W_p, g_p, be_p, g_e, be_e, W_f, g_f, be_f)


def _sc_scatter(h2, edge_index, zeros):
    mesh = plsc.VectorSubcoreMesh(core_axis_name="c", subcore_axis_name="s")

    @functools.partial(
        pl.kernel,
        out_type=(jax.ShapeDtypeStruct((N, H), jnp.float32),
                  jax.ShapeDtypeStruct((N, H), jnp.float32)),
        mesh=mesh,
        scratch_types=[
            pltpu.VMEM((CHUNK,), jnp.int32),
            pltpu.VMEM((CHUNK,), jnp.int32),
            pltpu.VMEM((CHUNK, H), jnp.float32),
            pltpu.VMEM_SHARED((N, H), jnp.float32),
            pltpu.SemaphoreType.DMA,
        ],
        compiler_params=pltpu.CompilerParams(use_tc_tiling_on_sc=False),
    )
    def k(h2_hbm, ei_hbm, zeros_hbm, out0_hbm, out1_hbm,
          idx_v, dst_v, rows_v, agg_sh, sem):
        c = lax.axis_index("c")
        s = lax.axis_index("s")
        w = c * 16 + s

        def _row_copy(src_ref, dst_ref):
            off = pl.multiple_of(s * ROWS_PER_S, 8)

            @pl.when(s < 15)
            def _():
                pltpu.sync_copy(src_ref.at[pl.ds(off, ROWS_PER_S)],
                                dst_ref.at[pl.ds(off, ROWS_PER_S)])

            @pl.when(s == 15)
            def _():
                pltpu.sync_copy(src_ref.at[pl.ds(15 * ROWS_PER_S, ROWS_LAST)],
                                dst_ref.at[pl.ds(15 * ROWS_PER_S, ROWS_LAST)])

        _row_copy(zeros_hbm, agg_sh)
        plsc.subcore_barrier()

        def step(t):
            base = pl.multiple_of(t * CHUNK, 8)
            pltpu.sync_copy(ei_hbm.at[0, pl.ds(base, CHUNK)], idx_v)
            pltpu.async_copy(h2_hbm.at[idx_v], rows_v, sem).wait()
            pltpu.sync_copy(ei_hbm.at[1, pl.ds(base, CHUNK)], dst_v)
            pltpu.sync_copy(rows_v, agg_sh.at[dst_v], add=True)

        def body(j, carry):
            step(j * NW + w)
            return carry
        lax.fori_loop(0, NCHUNK // NW, body, 0)

        plsc.subcore_barrier()

        @pl.when(c == 0)
        def _():
            _row_copy(agg_sh, out0_hbm)

        @pl.when(c == 1)
        def _():
            _row_copy(agg_sh, out1_hbm)

    return k(h2, edge_index, zeros)


def _out_body(agg0_ref, agg1_ref, h2_ref, wl_ref, wr_ref, gc_ref, bec_ref,
              wo_ref, bo_ref, z_ref, out_ref):
    agg = agg0_ref[...] + agg1_ref[...]
    h2 = h2_ref[...]
    z0 = (jnp.dot(agg, wl_ref[...], preferred_element_type=jnp.float32)
          + jnp.dot(h2, wr_ref[...], preferred_element_type=jnp.float32))
    z = _gelu(_bn_cols(z0, gc_ref[...].reshape(1, H),
                       bec_ref[...].reshape(1, H)))
    z_ref[...] = z
    out_ref[...] = jnp.maximum(
        jnp.dot(z, wo_ref[...], preferred_element_type=jnp.float32)
        + bo_ref[...].reshape(1, 1), 0.0)


def _out_stage(agg0, agg1, h2, W_l, W_r, g_c, be_c, W_o, b_o):
    return pl.pallas_call(
        _out_body,
        out_shape=(jax.ShapeDtypeStruct((N, H), jnp.float32),
                   jax.ShapeDtypeStruct((N, 1), jnp.float32)),
    )(agg0, agg1, h2, W_l, W_r, g_c, be_c, W_o, b_o)


def kernel(x, seq, pause, edge_index, W_x, b_x, g_x, be_x, W_p, b_p, g_p, be_p,
           W_e, b_e, g_e, be_e, W_f, b_f, g_f, be_f, W_l, b_l, W_r, g_c, be_c,
           W_o, b_o):
    A = _encoder_gemm(seq, W_e)

    h2 = _mid_stage(
        A, x, pause.reshape(N, 1),
        W_x, g_x, be_x, W_p, g_p, be_p, g_e, be_e, W_f, g_f, be_f)

    zeros = jnp.zeros((N, H), jnp.float32)
    agg0, agg1 = _sc_scatter(h2, edge_index, zeros)

    z, out = _out_stage(agg0, agg1, h2, W_l, W_r, g_c, be_c, W_o, b_o)
    return (out, z)
